# baseline (speedup 1.0000x reference)
import functools

import jax
import jax.numpy as jnp
from jax import lax
from jax.experimental import pallas as pl
from jax.experimental.pallas import tpu as pltpu
from jax.experimental.pallas import tpu_sc as plsc

N = 100000
VEC = 256
NROW = 120
NC = 2
NS = 16
NW = NC * NS
LANE = 16
CH = 112
NCH = 28
BPW = CH * NCH
BTOT = BPW * NW
NBUF = 3
RATIO = 4
BUILD_PHASE = 3

_LASTBASE = (NW - 1) * BPW
FULL = (N - _LASTBASE) // CH
PART = N - _LASTBASE - FULL * CH


def _is_build(c):
    return c % RATIO == BUILD_PHASE


@functools.partial(
    pl.kernel,
    out_type=jax.ShapeDtypeStruct((N, VEC), jnp.float32),
    mesh=plsc.VectorSubcoreMesh(core_axis_name="c", subcore_axis_name="s"),
    compiler_params=pltpu.CompilerParams(needs_layout_passes=False),
    scratch_types=[
        pltpu.VMEM((NROW * VEC,), jnp.float32),
        pltpu.VMEM((BPW,), jnp.int32),
        pltpu.VMEM((CH, VEC), jnp.float32),
        pltpu.VMEM((CH, VEC), jnp.float32),
        pltpu.VMEM((CH, VEC), jnp.float32),
        pltpu.SemaphoreType.DMA,
        pltpu.SemaphoreType.DMA,
    ],
)
def _emb_lookup(x_hbm, table_hbm, tflat_hbm, out_hbm, table_v, idx_v,
                rows_a, rows_b, rows_c, gsem, osem):
    wid = lax.axis_index("s") * NC + lax.axis_index("c")
    base = wid * BPW
    pltpu.sync_copy(x_hbm.at[pl.ds(base, BPW)], idx_v)
    pltpu.sync_copy(tflat_hbm, table_v)

    bufs = (rows_a, rows_b, rows_c)
    not_last = wid != NW - 1
    iota = lax.iota(jnp.int32, LANE)

    def gather(c):
        return pltpu.async_copy(
            table_hbm.at[idx_v.at[pl.ds(c * CH, CH)]], bufs[c % NBUF], gsem)

    def build(c):
        buf = bufs[c % NBUF]

        def row_body(r, _):
            av = iota * 0 + (c * CH + r)
            bidx = plsc.load_gather(idx_v, [av])
            rowb = bidx * VEC + iota
            vals = [plsc.load_gather(table_v, [rowb + j * LANE])
                    for j in range(VEC // LANE)]
            for j in range(VEC // LANE):
                buf[r, pl.ds(j * LANE, LANE)] = vals[j]
            return _

        lax.fori_loop(0, CH, row_body, 0, unroll=2)

    def store_copy(c):
        return pltpu.make_async_copy(
            bufs[c % NBUF], out_hbm.at[pl.ds(base + c * CH, CH)], osem)

    def issue_store(c):
        if c < FULL:
            store_copy(c).start()
        else:
            @pl.when(not_last)
            def _():
                store_copy(c).start()
            if c == FULL:
                @pl.when(jnp.logical_not(not_last))
                def _():
                    pltpu.sync_copy(
                        bufs[c % NBUF].at[pl.ds(0, PART)],
                        out_hbm.at[pl.ds(base + c * CH, PART)])

    def wait_store(c):
        if c < FULL:
            store_copy(c).wait()
        else:
            @pl.when(not_last)
            def _():
                store_copy(c).wait()

    g = [None] * NCH
    g[0] = gather(0)
    g[1] = gather(1)
    for c in range(NCH):
        if _is_build(c):
            build(c)
        else:
            g[c].wait()
        issue_store(c)
        if c + 2 < NCH:
            if c >= 1:
                wait_store(c - 1)
            if not _is_build(c + 2):
                g[c + 2] = gather(c + 2)
    wait_store(NCH - 3)
    wait_store(NCH - 2)
    wait_store(NCH - 1)


REP = 64


def kernel(x, table):
    idx = x.astype(jnp.int32)
    idx_p = jnp.concatenate([idx, jnp.zeros((BTOT - N,), jnp.int32)])
    i = jnp.arange(BTOT, dtype=jnp.int32)
    is_build = (i // CH) % RATIO == BUILD_PHASE
    shift = jnp.where(is_build, 0, (i % REP) * table.shape[0])
    table_rep = jnp.tile(table, (REP, 1))
    return _emb_lookup(idx_p + shift, table_rep, table.reshape(NROW * VEC))


# speedup vs baseline: 1.0091x; 1.0091x over previous
import functools

import jax
import jax.numpy as jnp
from jax import lax
from jax.experimental import pallas as pl
from jax.experimental.pallas import tpu as pltpu
from jax.experimental.pallas import tpu_sc as plsc

N = 100000
VEC = 256
NC = 2
NS = 16
NW = NC * NS
CH = 136
NCH = 23
BPW = CH * NCH
BTOT = BPW * NW
TAIL = N - (NW - 1) * BPW - (NCH - 1) * CH


@functools.partial(
    pl.kernel,
    out_type=jax.ShapeDtypeStruct((N, VEC), jnp.float32),
    mesh=plsc.VectorSubcoreMesh(core_axis_name="c", subcore_axis_name="s"),
    scratch_types=[
        pltpu.VMEM((BPW,), jnp.int32),
        pltpu.VMEM((CH, VEC), jnp.float32),
        pltpu.VMEM((CH, VEC), jnp.float32),
        pltpu.VMEM((CH, VEC), jnp.float32),
        pltpu.SemaphoreType.DMA,
        pltpu.SemaphoreType.DMA,
    ],
)
def _emb_lookup(x_hbm, table_hbm, out_hbm, idx_v, rows_a, rows_b, rows_c,
                gsem, osem):
    wid = lax.axis_index("s") * NC + lax.axis_index("c")
    base = wid * BPW
    pltpu.sync_copy(x_hbm.at[pl.ds(base, BPW)], idx_v)

    bufs = (rows_a, rows_b, rows_c)

    H = 72

    class _Pair:
        def __init__(self, a, b):
            self.a, self.b = a, b

        def wait(self):
            self.a.wait()
            self.b.wait()

    def gather(c):
        buf = bufs[c % 3]
        a = pltpu.async_copy(
            table_hbm.at[idx_v.at[pl.ds(c * CH, H)]], buf.at[pl.ds(0, H)],
            gsem)
        b = pltpu.async_copy(
            table_hbm.at[idx_v.at[pl.ds(c * CH + H, CH - H)]],
            buf.at[pl.ds(H, CH - H)], gsem)
        return _Pair(a, b)

    def store(c):
        return pltpu.async_copy(
            bufs[c % 3], out_hbm.at[pl.ds(base + c * CH, CH)], osem)

    g = [None] * NCH
    s = [None] * NCH
    g[0] = gather(0)
    g[1] = gather(1)
    for c in range(NCH - 1):
        g[c].wait()
        s[c] = store(c)
        if c + 2 < NCH:
            if c >= 1:
                s[c - 1].wait()
            g[c + 2] = gather(c + 2)
    s[NCH - 3].wait()
    s[NCH - 2].wait()

    g[NCH - 1].wait()
    last = NCH - 1
    is_tail = wid == NW - 1

    @pl.when(is_tail)
    def _():
        pltpu.sync_copy(bufs[last % 3].at[pl.ds(0, TAIL)],
                        out_hbm.at[pl.ds(base + last * CH, TAIL)])

    @pl.when(jnp.logical_not(is_tail))
    def _():
        pltpu.sync_copy(bufs[last % 3],
                        out_hbm.at[pl.ds(base + last * CH, CH)])


REP = 64


def kernel(x, table):
    idx = x.astype(jnp.int32)
    idx_p = jnp.concatenate([idx, jnp.zeros((BTOT - N,), jnp.int32)])
    shift = (jnp.arange(BTOT, dtype=jnp.int32) % REP) * table.shape[0]
    table_rep = jnp.tile(table, (REP, 1))
    return _emb_lookup(idx_p + shift, table_rep)
